# baseline (speedup 1.0000x reference)
import jax
import jax.numpy as jnp
from jax.experimental import pallas as pl
from jax.experimental.pallas import tpu as pltpu

N = 10000
N_IN = 128
N_H = 64
BM = 400
BF = 2000
F = N // BF
MI = N // BM

_DN_T = (((1,), (1,)), ((), ()))


def _dgi_kernel(seq1_ref, seq2_ref, adj_ref, wfc_ref, b_ref, a_ref,
                mskblk_ref, sb1_ref, sb2_ref, wbil_ref, bbil_ref,
                out_ref, fts_ref, h_ref, csum_ref, msksum_ref):
    g = pl.program_id(0)
    num_blocks = pl.num_programs(0)

    @pl.when(g == 0)
    def _init():
        csum_ref[...] = jnp.zeros_like(csum_ref)
        msksum_ref[0, 0] = 0.0

    @pl.when(g < F)
    def _fts():
        fts_ref[pl.ds(g * BF, BF), :N_H] = jax.lax.dot_general(
            seq1_ref[...], wfc_ref[...], _DN_T,
            preferred_element_type=jnp.float32).astype(jnp.bfloat16)
        fts_ref[pl.ds(g * BF, BF), N_H:] = jax.lax.dot_general(
            seq2_ref[...], wfc_ref[...], _DN_T,
            preferred_element_type=jnp.float32).astype(jnp.bfloat16)

    @pl.when(g >= F)
    def _adj_step():
        m = g - F
        out = jnp.dot(adj_ref[...].astype(jnp.bfloat16), fts_ref[...],
                      preferred_element_type=jnp.float32)
        b2 = jnp.concatenate([b_ref[...], b_ref[...]], axis=1)
        out = out + b2
        a = a_ref[0, 0]
        h = jnp.where(out > 0, out, a * out)
        h_ref[pl.ds(m * BM, BM), :] = h

        msk_blk = mskblk_ref[0]
        csum_ref[...] += jnp.dot(msk_blk, h,
                                 preferred_element_type=jnp.float32)
        msksum_ref[0, 0] += jnp.sum(msk_blk)

    @pl.when(g == num_blocks - 1)
    def _finish():
        c = jax.nn.sigmoid(csum_ref[:, :N_H] / msksum_ref[0, 0])
        v = jax.lax.dot_general(c, wbil_ref[...], _DN_T,
                                preferred_element_type=jnp.float32)
        h1 = h_ref[:, :N_H]
        h2 = h_ref[:, N_H:]
        sc1 = jax.lax.dot_general(v, h1, _DN_T,
                                  preferred_element_type=jnp.float32)
        sc2 = jax.lax.dot_general(v, h2, _DN_T,
                                  preferred_element_type=jnp.float32)
        b = bbil_ref[0, 0]
        out_ref[:, :N] = sc1 + b + sb1_ref[...]
        out_ref[:, N:] = sc2 + b + sb2_ref[...]


def kernel(seq1, seq2, adj, sparse, msk, samp_bias1, samp_bias2,
           W_fc, b_gcn, prelu_a, W_bil, b_bil):
    del sparse
    seq1_2d = seq1.reshape(N, N_IN)
    seq2_2d = seq2.reshape(N, N_IN)
    adj_2d = adj.reshape(N, N)
    b_2d = b_gcn.reshape(1, N_H)
    a2 = prelu_a.reshape(1, 1)
    msk_blocks = msk.reshape(N // BM, 1, BM)
    wbil_2d = W_bil.reshape(N_H, N_H)
    bbil_2d = b_bil.reshape(1, 1)

    grid = (F + MI,)
    full = lambda g: (0, 0)
    out = pl.pallas_call(
        _dgi_kernel,
        grid=grid,
        in_specs=[
            pl.BlockSpec((BF, N_IN),
                         lambda g: (jnp.minimum(g, F - 1), 0)),
            pl.BlockSpec((BF, N_IN),
                         lambda g: (jnp.minimum(g, F - 1), 0)),
            pl.BlockSpec((BM, N),
                         lambda g: (jnp.maximum(g - F, 0), 0)),
            pl.BlockSpec((N_H, N_IN), full),
            pl.BlockSpec((1, N_H), full),
            pl.BlockSpec((1, 1), full),
            pl.BlockSpec((1, 1, BM),
                         lambda g: (jnp.maximum(g - F, 0), 0, 0)),
            pl.BlockSpec((1, N), full),
            pl.BlockSpec((1, N), full),
            pl.BlockSpec((N_H, N_H), full),
            pl.BlockSpec((1, 1), full),
        ],
        out_specs=pl.BlockSpec((1, 2 * N), full),
        out_shape=jax.ShapeDtypeStruct((1, 2 * N), jnp.float32),
        scratch_shapes=[
            pltpu.VMEM((N, 2 * N_H), jnp.bfloat16),
            pltpu.VMEM((N, 2 * N_H), jnp.float32),
            pltpu.VMEM((1, 2 * N_H), jnp.float32),
            pltpu.SMEM((1, 1), jnp.float32),
        ],
        compiler_params=pltpu.CompilerParams(
            dimension_semantics=("arbitrary",),
            vmem_limit_bytes=64 * 1024 * 1024,
        ),
    )(seq1_2d, seq2_2d, adj_2d, W_fc, b_2d, a2, msk_blocks,
      samp_bias1, samp_bias2, wbil_2d, bbil_2d)

    return out


# speedup vs baseline: 1.0129x; 1.0129x over previous
import jax
import jax.numpy as jnp
from jax.experimental import pallas as pl
from jax.experimental.pallas import tpu as pltpu

N = 10000
N_IN = 128
N_H = 64
BM = 400
BF = 2000
F = N // BF
MI = N // BM

_DN_T = (((1,), (1,)), ((), ()))


def _dgi_kernel(seq1_ref, seq2_ref, adj_ref, wfc_ref, b_ref, a_ref,
                msk_ref, sb1_ref, sb2_ref, wbil_ref, bbil_ref,
                out_ref, fts_ref, h_ref):
    g = pl.program_id(0)
    num_blocks = pl.num_programs(0)

    @pl.when(g < F)
    def _fts():
        fts_ref[pl.ds(g * BF, BF), :N_H] = jax.lax.dot_general(
            seq1_ref[...], wfc_ref[...], _DN_T,
            preferred_element_type=jnp.float32).astype(jnp.bfloat16)
        fts_ref[pl.ds(g * BF, BF), N_H:] = jax.lax.dot_general(
            seq2_ref[...], wfc_ref[...], _DN_T,
            preferred_element_type=jnp.float32).astype(jnp.bfloat16)

    @pl.when(g >= F)
    def _adj_step():
        m = g - F
        out = jnp.dot(adj_ref[...].astype(jnp.bfloat16), fts_ref[...],
                      preferred_element_type=jnp.float32)
        b2 = jnp.concatenate([b_ref[...], b_ref[...]], axis=1)
        out = out + b2
        a = a_ref[0, 0]
        h = jnp.where(out > 0, out, a * out)
        h_ref[pl.ds(m * BM, BM), :] = h

    @pl.when(g == num_blocks - 1)
    def _finish():
        csum = jnp.dot(msk_ref[...], h_ref[...],
                       preferred_element_type=jnp.float32)
        msk_total = jnp.sum(msk_ref[...])
        c = jax.nn.sigmoid(csum[:, :N_H] / msk_total)
        v = jax.lax.dot_general(c, wbil_ref[...], _DN_T,
                                preferred_element_type=jnp.float32)
        h1 = h_ref[:, :N_H]
        h2 = h_ref[:, N_H:]
        sc1 = jax.lax.dot_general(v, h1, _DN_T,
                                  preferred_element_type=jnp.float32)
        sc2 = jax.lax.dot_general(v, h2, _DN_T,
                                  preferred_element_type=jnp.float32)
        b = bbil_ref[0, 0]
        out_ref[:, :N] = sc1 + b + sb1_ref[...]
        out_ref[:, N:] = sc2 + b + sb2_ref[...]


def kernel(seq1, seq2, adj, sparse, msk, samp_bias1, samp_bias2,
           W_fc, b_gcn, prelu_a, W_bil, b_bil):
    del sparse
    seq1_2d = seq1.reshape(N, N_IN)
    seq2_2d = seq2.reshape(N, N_IN)
    adj_2d = adj.reshape(N, N)
    b_2d = b_gcn.reshape(1, N_H)
    a2 = prelu_a.reshape(1, 1)
    msk_2d = msk.reshape(1, N)
    wbil_2d = W_bil.reshape(N_H, N_H)
    bbil_2d = b_bil.reshape(1, 1)

    grid = (F + MI,)
    full = lambda g: (0, 0)
    out = pl.pallas_call(
        _dgi_kernel,
        grid=grid,
        in_specs=[
            pl.BlockSpec((BF, N_IN),
                         lambda g: (jnp.minimum(g, F - 1), 0)),
            pl.BlockSpec((BF, N_IN),
                         lambda g: (jnp.minimum(g, F - 1), 0)),
            pl.BlockSpec((BM, N),
                         lambda g: (jnp.maximum(g - F, 0), 0)),
            pl.BlockSpec((N_H, N_IN), full),
            pl.BlockSpec((1, N_H), full),
            pl.BlockSpec((1, 1), full),
            pl.BlockSpec((1, N), full),
            pl.BlockSpec((1, N), full),
            pl.BlockSpec((1, N), full),
            pl.BlockSpec((N_H, N_H), full),
            pl.BlockSpec((1, 1), full),
        ],
        out_specs=pl.BlockSpec((1, 2 * N), full),
        out_shape=jax.ShapeDtypeStruct((1, 2 * N), jnp.float32),
        scratch_shapes=[
            pltpu.VMEM((N, 2 * N_H), jnp.bfloat16),
            pltpu.VMEM((N, 2 * N_H), jnp.float32),
        ],
        compiler_params=pltpu.CompilerParams(
            dimension_semantics=("arbitrary",),
            vmem_limit_bytes=64 * 1024 * 1024,
        ),
    )(seq1_2d, seq2_2d, adj_2d, W_fc, b_2d, a2, msk_2d,
      samp_bias1, samp_bias2, wbil_2d, bbil_2d)

    return out


# speedup vs baseline: 1.0131x; 1.0002x over previous
import jax
import jax.numpy as jnp
from jax.experimental import pallas as pl
from jax.experimental.pallas import tpu as pltpu

N = 10000
N_IN = 128
N_H = 64
BM = 400
BF = 2000
F = N // BF
MI = N // BM

_DN_T = (((1,), (1,)), ((), ()))


def _dgi_kernel(seq1_ref, seq2_ref, adj_ref, wfc_ref, b_ref, a_ref,
                msk_ref, sb1_ref, sb2_ref, wbil_ref, bbil_ref,
                out_ref, fts_ref, h_ref):
    g = pl.program_id(0)
    num_blocks = pl.num_programs(0)

    @pl.when(g < F)
    def _fts():
        fts_ref[pl.ds(g * BF, BF), :N_H] = jax.lax.dot_general(
            seq1_ref[...], wfc_ref[...], _DN_T,
            preferred_element_type=jnp.float32).astype(jnp.bfloat16)
        fts_ref[pl.ds(g * BF, BF), N_H:] = jax.lax.dot_general(
            seq2_ref[...], wfc_ref[...], _DN_T,
            preferred_element_type=jnp.float32).astype(jnp.bfloat16)

    @pl.when(g >= F)
    def _adj_step():
        m = g - F
        out = jnp.dot(adj_ref[...].astype(jnp.bfloat16), fts_ref[...],
                      preferred_element_type=jnp.float32)
        b2 = jnp.concatenate([b_ref[...], b_ref[...]], axis=1)
        out = out + b2
        a = a_ref[0, 0]
        h = jnp.where(out > 0, out, a * out)
        h_ref[pl.ds(m * BM, BM), :] = h.astype(jnp.bfloat16)

    @pl.when(g == num_blocks - 1)
    def _finish():
        csum = jnp.dot(msk_ref[...], h_ref[...],
                       preferred_element_type=jnp.float32)
        msk_total = jnp.sum(msk_ref[...])
        c = jax.nn.sigmoid(csum[:, :N_H] / msk_total)
        v = jax.lax.dot_general(c, wbil_ref[...], _DN_T,
                                preferred_element_type=jnp.float32)
        h1 = h_ref[:, :N_H]
        h2 = h_ref[:, N_H:]
        sc1 = jax.lax.dot_general(v, h1, _DN_T,
                                  preferred_element_type=jnp.float32)
        sc2 = jax.lax.dot_general(v, h2, _DN_T,
                                  preferred_element_type=jnp.float32)
        b = bbil_ref[0, 0]
        out_ref[:, :N] = sc1 + b + sb1_ref[...]
        out_ref[:, N:] = sc2 + b + sb2_ref[...]


def kernel(seq1, seq2, adj, sparse, msk, samp_bias1, samp_bias2,
           W_fc, b_gcn, prelu_a, W_bil, b_bil):
    del sparse
    seq1_2d = seq1.reshape(N, N_IN)
    seq2_2d = seq2.reshape(N, N_IN)
    adj_2d = adj.reshape(N, N)
    b_2d = b_gcn.reshape(1, N_H)
    a2 = prelu_a.reshape(1, 1)
    msk_2d = msk.reshape(1, N)
    wbil_2d = W_bil.reshape(N_H, N_H)
    bbil_2d = b_bil.reshape(1, 1)

    grid = (F + MI,)
    full = lambda g: (0, 0)
    out = pl.pallas_call(
        _dgi_kernel,
        grid=grid,
        in_specs=[
            pl.BlockSpec((BF, N_IN),
                         lambda g: (jnp.minimum(g, F - 1), 0)),
            pl.BlockSpec((BF, N_IN),
                         lambda g: (jnp.minimum(g, F - 1), 0)),
            pl.BlockSpec((BM, N),
                         lambda g: (jnp.maximum(g - F, 0), 0)),
            pl.BlockSpec((N_H, N_IN), full),
            pl.BlockSpec((1, N_H), full),
            pl.BlockSpec((1, 1), full),
            pl.BlockSpec((1, N), full),
            pl.BlockSpec((1, N), full),
            pl.BlockSpec((1, N), full),
            pl.BlockSpec((N_H, N_H), full),
            pl.BlockSpec((1, 1), full),
        ],
        out_specs=pl.BlockSpec((1, 2 * N), full),
        out_shape=jax.ShapeDtypeStruct((1, 2 * N), jnp.float32),
        scratch_shapes=[
            pltpu.VMEM((N, 2 * N_H), jnp.bfloat16),
            pltpu.VMEM((N, 2 * N_H), jnp.bfloat16),
        ],
        compiler_params=pltpu.CompilerParams(
            dimension_semantics=("arbitrary",),
            vmem_limit_bytes=64 * 1024 * 1024,
        ),
    )(seq1_2d, seq2_2d, adj_2d, W_fc, b_2d, a2, msk_2d,
      samp_bias1, samp_bias2, wbil_2d, bbil_2d)

    return out


# speedup vs baseline: 1.0143x; 1.0012x over previous
import jax
import jax.numpy as jnp
from jax.experimental import pallas as pl
from jax.experimental.pallas import tpu as pltpu

N = 10000
N_IN = 128
N_H = 64
BM = 400
BF = 2000
F = N // BF
MI = N // BM

_DN_T = (((1,), (1,)), ((), ()))


def _dgi_kernel(seq1_ref, seq2_ref, adj_ref, wfc_ref, b_ref, a_ref,
                msk_ref, sb1_ref, sb2_ref, wbil_ref, bbil_ref,
                out_ref, fts_ref, h_ref):
    g = pl.program_id(0)
    num_blocks = pl.num_programs(0)

    @pl.when(g < F)
    def _fts():
        fts_ref[pl.ds(g * BF, BF), :N_H] = jax.lax.dot_general(
            seq1_ref[...], wfc_ref[...], _DN_T,
            preferred_element_type=jnp.float32).astype(jnp.bfloat16)
        fts_ref[pl.ds(g * BF, BF), N_H:] = jax.lax.dot_general(
            seq2_ref[...], wfc_ref[...], _DN_T,
            preferred_element_type=jnp.float32).astype(jnp.bfloat16)

    @pl.when(g >= F)
    def _adj_step():
        m = g - F
        out = jnp.dot(adj_ref[...].astype(jnp.bfloat16), fts_ref[...],
                      preferred_element_type=jnp.float32)
        b2 = jnp.concatenate([b_ref[...], b_ref[...]], axis=1)
        out = out + b2
        a = a_ref[0, 0]
        h = jnp.where(out > 0, out, a * out)
        h_ref[pl.ds(m * BM, BM), :] = h

    @pl.when(g == num_blocks - 1)
    def _finish():
        csum = jnp.dot(msk_ref[...], h_ref[:, :N_H],
                       preferred_element_type=jnp.float32)
        msk_total = jnp.sum(msk_ref[...])
        c = jax.nn.sigmoid(csum / msk_total)
        v = jax.lax.dot_general(c, wbil_ref[...], _DN_T,
                                preferred_element_type=jnp.float32)
        z = jnp.zeros_like(v)
        v2 = jnp.concatenate([jnp.concatenate([v, z], axis=1),
                              jnp.concatenate([z, v], axis=1)], axis=0)
        sc = jax.lax.dot_general(v2, h_ref[...], _DN_T,
                                 preferred_element_type=jnp.float32)
        b = bbil_ref[0, 0]
        out_ref[:, :N] = sc[0:1, :] + b + sb1_ref[...]
        out_ref[:, N:] = sc[1:2, :] + b + sb2_ref[...]


def kernel(seq1, seq2, adj, sparse, msk, samp_bias1, samp_bias2,
           W_fc, b_gcn, prelu_a, W_bil, b_bil):
    del sparse
    seq1_2d = seq1.reshape(N, N_IN)
    seq2_2d = seq2.reshape(N, N_IN)
    adj_2d = adj.reshape(N, N)
    b_2d = b_gcn.reshape(1, N_H)
    a2 = prelu_a.reshape(1, 1)
    msk_2d = msk.reshape(1, N)
    wbil_2d = W_bil.reshape(N_H, N_H)
    bbil_2d = b_bil.reshape(1, 1)

    grid = (F + MI,)
    full = lambda g: (0, 0)
    out = pl.pallas_call(
        _dgi_kernel,
        grid=grid,
        in_specs=[
            pl.BlockSpec((BF, N_IN),
                         lambda g: (jnp.minimum(g, F - 1), 0)),
            pl.BlockSpec((BF, N_IN),
                         lambda g: (jnp.minimum(g, F - 1), 0)),
            pl.BlockSpec((BM, N),
                         lambda g: (jnp.maximum(g - F, 0), 0)),
            pl.BlockSpec((N_H, N_IN), full),
            pl.BlockSpec((1, N_H), full),
            pl.BlockSpec((1, 1), full),
            pl.BlockSpec((1, N), full),
            pl.BlockSpec((1, N), full),
            pl.BlockSpec((1, N), full),
            pl.BlockSpec((N_H, N_H), full),
            pl.BlockSpec((1, 1), full),
        ],
        out_specs=pl.BlockSpec((1, 2 * N), full),
        out_shape=jax.ShapeDtypeStruct((1, 2 * N), jnp.float32),
        scratch_shapes=[
            pltpu.VMEM((N, 2 * N_H), jnp.bfloat16),
            pltpu.VMEM((N, 2 * N_H), jnp.float32),
        ],
        compiler_params=pltpu.CompilerParams(
            dimension_semantics=("arbitrary",),
            vmem_limit_bytes=64 * 1024 * 1024,
        ),
    )(seq1_2d, seq2_2d, adj_2d, W_fc, b_2d, a2, msk_2d,
      samp_bias1, samp_bias2, wbil_2d, bbil_2d)

    return out


# speedup vs baseline: 1.0188x; 1.0045x over previous
import jax
import jax.numpy as jnp
from jax.experimental import pallas as pl
from jax.experimental.pallas import tpu as pltpu

N = 10000
N_IN = 128
N_H = 64
BM = 400
BF = 5000
F = N // BF
MI = N // BM

_DN_T = (((1,), (1,)), ((), ()))


def _dgi_kernel(seq1_ref, seq2_ref, adj_ref, wfc_ref, b_ref, a_ref,
                msk_ref, sb1_ref, sb2_ref, wbil_ref, bbil_ref,
                out_ref, fts_ref, h_ref):
    g = pl.program_id(0)
    num_blocks = pl.num_programs(0)

    @pl.when(g < F)
    def _fts():
        fts_ref[pl.ds(g * BF, BF), :N_H] = jax.lax.dot_general(
            seq1_ref[...], wfc_ref[...], _DN_T,
            preferred_element_type=jnp.float32).astype(jnp.bfloat16)
        fts_ref[pl.ds(g * BF, BF), N_H:] = jax.lax.dot_general(
            seq2_ref[...], wfc_ref[...], _DN_T,
            preferred_element_type=jnp.float32).astype(jnp.bfloat16)

    @pl.when(g >= F)
    def _adj_step():
        m = g - F
        out = jnp.dot(adj_ref[...].astype(jnp.bfloat16), fts_ref[...],
                      preferred_element_type=jnp.float32)
        b2 = jnp.concatenate([b_ref[...], b_ref[...]], axis=1)
        out = out + b2
        a = a_ref[0, 0]
        h = jnp.where(out > 0, out, a * out)
        h_ref[pl.ds(m * BM, BM), :] = h

    @pl.when(g == num_blocks - 1)
    def _finish():
        csum = jnp.dot(msk_ref[...], h_ref[:, :N_H],
                       preferred_element_type=jnp.float32)
        msk_total = jnp.sum(msk_ref[...])
        c = jax.nn.sigmoid(csum / msk_total)
        v = jax.lax.dot_general(c, wbil_ref[...], _DN_T,
                                preferred_element_type=jnp.float32)
        z = jnp.zeros_like(v)
        v2 = jnp.concatenate([jnp.concatenate([v, z], axis=1),
                              jnp.concatenate([z, v], axis=1)], axis=0)
        sc = jax.lax.dot_general(v2, h_ref[...], _DN_T,
                                 preferred_element_type=jnp.float32)
        b = bbil_ref[0, 0]
        out_ref[:, :N] = sc[0:1, :] + b + sb1_ref[...]
        out_ref[:, N:] = sc[1:2, :] + b + sb2_ref[...]


def kernel(seq1, seq2, adj, sparse, msk, samp_bias1, samp_bias2,
           W_fc, b_gcn, prelu_a, W_bil, b_bil):
    del sparse
    seq1_2d = seq1.reshape(N, N_IN)
    seq2_2d = seq2.reshape(N, N_IN)
    adj_2d = adj.reshape(N, N)
    b_2d = b_gcn.reshape(1, N_H)
    a2 = prelu_a.reshape(1, 1)
    msk_2d = msk.reshape(1, N)
    wbil_2d = W_bil.reshape(N_H, N_H)
    bbil_2d = b_bil.reshape(1, 1)

    grid = (F + MI,)
    full = lambda g: (0, 0)
    out = pl.pallas_call(
        _dgi_kernel,
        grid=grid,
        in_specs=[
            pl.BlockSpec((BF, N_IN),
                         lambda g: (jnp.minimum(g, F - 1), 0)),
            pl.BlockSpec((BF, N_IN),
                         lambda g: (jnp.minimum(g, F - 1), 0)),
            pl.BlockSpec((BM, N),
                         lambda g: (jnp.maximum(g - F, 0), 0)),
            pl.BlockSpec((N_H, N_IN), full),
            pl.BlockSpec((1, N_H), full),
            pl.BlockSpec((1, 1), full),
            pl.BlockSpec((1, N), full),
            pl.BlockSpec((1, N), full),
            pl.BlockSpec((1, N), full),
            pl.BlockSpec((N_H, N_H), full),
            pl.BlockSpec((1, 1), full),
        ],
        out_specs=pl.BlockSpec((1, 2 * N), full),
        out_shape=jax.ShapeDtypeStruct((1, 2 * N), jnp.float32),
        scratch_shapes=[
            pltpu.VMEM((N, 2 * N_H), jnp.bfloat16),
            pltpu.VMEM((N, 2 * N_H), jnp.float32),
        ],
        compiler_params=pltpu.CompilerParams(
            dimension_semantics=("arbitrary",),
            vmem_limit_bytes=64 * 1024 * 1024,
        ),
    )(seq1_2d, seq2_2d, adj_2d, W_fc, b_2d, a2, msk_2d,
      samp_bias1, samp_bias2, wbil_2d, bbil_2d)

    return out
